# baseline (speedup 1.0000x reference)
import functools

import jax
import jax.numpy as jnp
from jax import lax
from jax.experimental import pallas as pl
from jax.experimental.pallas import tpu as pltpu
from jax.experimental.pallas import tpu_sc as plsc

_D = 16
_B = 16384
_H = 50
_C = 20
_BETA = 0.5
_N = 1000001

_HP = 56
_CPD = 24
_OP = 32

_NC = 2
_NS = 16
_NW = _NC * _NS
_RPW = _B // _NW


def _sc_body(hidx_hbm, cidx_hbm, wh_hbm, wct_hbm, out_hbm,
             hidx_v, cidx_v, out_v,
             hb0, cb0, hb1, cb1,
             hs0, cs0, hs1, cs1):
    wid = lax.axis_index("s") * _NC + lax.axis_index("c")
    base = wid * _RPW

    pltpu.sync_copy(hidx_hbm.at[pl.ds(base * _HP, _RPW * _HP)], hidx_v)
    pltpu.sync_copy(cidx_hbm.at[pl.ds(base * _CPD, _RPW * _CPD)], cidx_v)

    hbufs = (hb0, hb1)
    cbufs = (cb0, cb1)
    hsems = (hs0, hs1)
    csems = (cs0, cs1)

    def issue(row, b):
        pltpu.async_copy(
            wh_hbm.at[hidx_v.at[pl.ds(row * _HP, _H)]], hbufs[b], hsems[b])
        pltpu.async_copy(
            wct_hbm.at[cidx_v.at[pl.ds(row * _CPD, _C)]], cbufs[b], csems[b])

    def wait(row, b):
        pltpu.make_async_copy(
            wh_hbm.at[hidx_v.at[pl.ds(row * _HP, _H)]], hbufs[b],
            hsems[b]).wait()
        pltpu.make_async_copy(
            wct_hbm.at[cidx_v.at[pl.ds(row * _CPD, _C)]], cbufs[b],
            csems[b]).wait()

    lanes = lax.iota(jnp.int32, _D)

    def compute(row, b):
        hb = hbufs[b]
        cb = cbufs[b]
        acc = hb[0]
        for h in range(1, _H):
            acc = acc + hb[h] * (_BETA ** h)
        ctx = [cb[c, 0:_D] for c in range(_C)]
        s = ctx[0]
        for c in range(1, _C):
            s = s + ctx[c]
        a = acc + s
        lo = jnp.zeros((_D,), jnp.float32)
        hi = jnp.zeros((_D,), jnp.float32)
        for c in range(_C):
            tgt = cb[c, _D:2 * _D]
            u = jnp.sum(tgt * (a - ctx[c]))
            if c < _D:
                lo = jnp.where(lanes == c, u, lo)
            else:
                hi = jnp.where(lanes == (c - _D), u, hi)
        out_v[pl.ds(row * _OP, _D)] = lo
        out_v[pl.ds(row * _OP + _D, _D)] = hi

    issue(0, 0)

    def body(i, carry):
        r = i * 2
        for b in range(2):
            row = r + b
            nxt = row + 1

            @pl.when(nxt < _RPW)
            def _():
                issue(nxt, 1 - b)

            wait(row, b)
            compute(row, b)
        return carry

    lax.fori_loop(0, _RPW // 2, body, 0, unroll=False)

    pltpu.sync_copy(out_v, out_hbm.at[pl.ds(base * _OP, _RPW * _OP)])


_sc_utilities = functools.partial(
    pl.kernel,
    out_type=jax.ShapeDtypeStruct((_B * _OP,), jnp.float32),
    mesh=plsc.VectorSubcoreMesh(core_axis_name="c", subcore_axis_name="s"),
    compiler_params=pltpu.CompilerParams(
        needs_layout_passes=False, use_tc_tiling_on_sc=False),
    scratch_types=[
        pltpu.VMEM((_RPW * _HP,), jnp.int32),
        pltpu.VMEM((_RPW * _CPD,), jnp.int32),
        pltpu.VMEM((_RPW * _OP,), jnp.float32),
        pltpu.VMEM((_H, _D), jnp.float32),
        pltpu.VMEM((_C, 2 * _D), jnp.float32),
        pltpu.VMEM((_H, _D), jnp.float32),
        pltpu.VMEM((_C, 2 * _D), jnp.float32),
        pltpu.SemaphoreType.DMA,
        pltpu.SemaphoreType.DMA,
        pltpu.SemaphoreType.DMA,
        pltpu.SemaphoreType.DMA,
    ],
)(_sc_body)


_RBLK = 512


def _repack16_body(xt_ref, o_ref):
    o_ref[...] = xt_ref[...].T


def _repack32_body(ct_ref, tt_ref, o_ref):
    o_ref[...] = jnp.concatenate([ct_ref[...].T, tt_ref[...].T], axis=1)


def _repack_tables(Wh, Wc, Wt):
    g = (_N + _RBLK - 1) // _RBLK
    wh = pl.pallas_call(
        _repack16_body,
        grid=(g,),
        in_specs=[pl.BlockSpec((_D, _RBLK), lambda i: (0, i))],
        out_specs=pl.BlockSpec((_RBLK, _D), lambda i: (i, 0)),
        out_shape=jax.ShapeDtypeStruct((_N, _D), jnp.float32),
    )(Wh.T)
    wct = pl.pallas_call(
        _repack32_body,
        grid=(g,),
        in_specs=[pl.BlockSpec((_D, _RBLK), lambda i: (0, i)),
                  pl.BlockSpec((_D, _RBLK), lambda i: (0, i))],
        out_specs=pl.BlockSpec((_RBLK, 2 * _D), lambda i: (i, 0)),
        out_shape=jax.ShapeDtypeStruct((_N, 2 * _D), jnp.float32),
    )(Wc.T, Wt.T)
    return wh, wct


def _softmax_body(u_ref, len_ref, o_ref):
    u = u_ref[...]
    ln = len_ref[...]
    col = lax.broadcasted_iota(jnp.int32, u.shape, 1)
    u = jnp.where((col >= ln) | (col >= _C), -jnp.inf, u)
    m = jnp.max(u, axis=1, keepdims=True)
    sh = u - m
    lse = jnp.log(jnp.sum(jnp.exp(sh), axis=1, keepdims=True))
    o_ref[...] = (sh - lse)[:, :_C]


_BLK = 2048


def _tc_logsoftmax(util, lens2d):
    return pl.pallas_call(
        _softmax_body,
        grid=(_B // _BLK,),
        in_specs=[
            pl.BlockSpec((_BLK, _OP), lambda i: (i, 0)),
            pl.BlockSpec((_BLK, 1), lambda i: (i, 0)),
        ],
        out_specs=pl.BlockSpec((_BLK, _C), lambda i: (i, 0)),
        out_shape=jax.ShapeDtypeStruct((_B, _C), jnp.float32),
    )(util, lens2d)


def kernel(histories, history_lengths, choice_sets, choice_set_lengths,
           Wh, Wc, Wt):
    del history_lengths
    hidx = jnp.pad(histories, ((0, 0), (0, _HP - _H))).reshape(-1)
    cidx = jnp.pad(choice_sets, ((0, 0), (0, _CPD - _C))).reshape(-1)
    wh, wct = _repack_tables(Wh, Wc, Wt)
    util = _sc_utilities(hidx, cidx, wh, wct).reshape(_B, _OP)
    return _tc_logsoftmax(util, choice_set_lengths.reshape(_B, 1))


# speedup vs baseline: 2.1475x; 2.1475x over previous
import functools

import jax
import jax.numpy as jnp
from jax import lax
from jax.experimental import pallas as pl
from jax.experimental.pallas import tpu as pltpu
from jax.experimental.pallas import tpu_sc as plsc

_D = 16
_B = 16384
_H = 50
_C = 20
_BETA = 0.5
_N = 1000001

_HP = 56
_CPD = 24
_OP = 32

_NC = 2
_NS = 16
_NW = _NC * _NS
_RPW = _B // _NW


def _sc_body(hidx_hbm, cidx_hbm, wh_hbm, wct_hbm, out_hbm,
             hidx_v, cidx_v, out_v,
             hb0, cb0, hb1, cb1,
             hs0, cs0, hs1, cs1):
    wid = lax.axis_index("s") * _NC + lax.axis_index("c")
    base = wid * _RPW

    pltpu.sync_copy(hidx_hbm.at[pl.ds(base * _HP, _RPW * _HP)], hidx_v)
    pltpu.sync_copy(cidx_hbm.at[pl.ds(base * _CPD, _RPW * _CPD)], cidx_v)

    hbufs = (hb0, hb1)
    cbufs = (cb0, cb1)
    hsems = (hs0, hs1)
    csems = (cs0, cs1)

    def issue(row, b):
        pltpu.async_copy(
            wh_hbm.at[hidx_v.at[pl.ds(row * _HP, _H)]], hbufs[b], hsems[b])
        pltpu.async_copy(
            wct_hbm.at[cidx_v.at[pl.ds(row * _CPD, _C)]], cbufs[b], csems[b])

    def wait(row, b):
        pltpu.make_async_copy(
            wh_hbm.at[hidx_v.at[pl.ds(row * _HP, _H)]], hbufs[b],
            hsems[b]).wait()
        pltpu.make_async_copy(
            wct_hbm.at[cidx_v.at[pl.ds(row * _CPD, _C)]], cbufs[b],
            csems[b]).wait()

    lanes = lax.iota(jnp.int32, _D)

    def compute(row, b):
        hb = hbufs[b]
        cb = cbufs[b]
        acc = hb[0]
        for h in range(1, _H):
            acc = acc + hb[h] * (_BETA ** h)
        ctx = [cb[c, 0:_D] for c in range(_C)]
        s = ctx[0]
        for c in range(1, _C):
            s = s + ctx[c]
        a = acc + s
        lo = jnp.zeros((_D,), jnp.float32)
        hi = jnp.zeros((_D,), jnp.float32)
        for c in range(_C):
            tgt = cb[c, _D:2 * _D]
            u = jnp.sum(tgt * (a - ctx[c]))
            if c < _D:
                lo = jnp.where(lanes == c, u, lo)
            else:
                hi = jnp.where(lanes == (c - _D), u, hi)
        out_v[pl.ds(row * _OP, _D)] = lo
        out_v[pl.ds(row * _OP + _D, _D)] = hi

    issue(0, 0)

    def body(i, carry):
        r = i * 2
        for b in range(2):
            row = r + b
            nxt = row + 1

            @pl.when(nxt < _RPW)
            def _():
                issue(nxt, 1 - b)

            wait(row, b)
            compute(row, b)
        return carry

    lax.fori_loop(0, _RPW // 2, body, 0, unroll=False)

    pltpu.sync_copy(out_v, out_hbm.at[pl.ds(base * _OP, _RPW * _OP)])


_sc_utilities = functools.partial(
    pl.kernel,
    out_type=jax.ShapeDtypeStruct((_B * _OP,), jnp.float32),
    mesh=plsc.VectorSubcoreMesh(core_axis_name="c", subcore_axis_name="s"),
    compiler_params=pltpu.CompilerParams(
        needs_layout_passes=False, use_tc_tiling_on_sc=False),
    scratch_types=[
        pltpu.VMEM((_RPW * _HP,), jnp.int32),
        pltpu.VMEM((_RPW * _CPD,), jnp.int32),
        pltpu.VMEM((_RPW * _OP,), jnp.float32),
        pltpu.VMEM((_H, _D), jnp.float32),
        pltpu.VMEM((_C, 2 * _D), jnp.float32),
        pltpu.VMEM((_H, _D), jnp.float32),
        pltpu.VMEM((_C, 2 * _D), jnp.float32),
        pltpu.SemaphoreType.DMA,
        pltpu.SemaphoreType.DMA,
        pltpu.SemaphoreType.DMA,
        pltpu.SemaphoreType.DMA,
    ],
)(_sc_body)


_RBLK = 8192


def _repack16_body(xt_ref, o_ref):
    o_ref[...] = xt_ref[...].T


def _repack32_body(ct_ref, tt_ref, o_ref):
    o_ref[...] = jnp.concatenate([ct_ref[...].T, tt_ref[...].T], axis=1)


def _repack_tables(Wh, Wc, Wt):
    g = (_N + _RBLK - 1) // _RBLK
    wh = pl.pallas_call(
        _repack16_body,
        grid=(g,),
        in_specs=[pl.BlockSpec((_D, _RBLK), lambda i: (0, i))],
        out_specs=pl.BlockSpec((_RBLK, _D), lambda i: (i, 0)),
        out_shape=jax.ShapeDtypeStruct((_N, _D), jnp.float32),
    )(Wh.T)
    wct = pl.pallas_call(
        _repack32_body,
        grid=(g,),
        in_specs=[pl.BlockSpec((_D, _RBLK), lambda i: (0, i)),
                  pl.BlockSpec((_D, _RBLK), lambda i: (0, i))],
        out_specs=pl.BlockSpec((_RBLK, 2 * _D), lambda i: (i, 0)),
        out_shape=jax.ShapeDtypeStruct((_N, 2 * _D), jnp.float32),
    )(Wc.T, Wt.T)
    return wh, wct


def _softmax_body(u_ref, len_ref, o_ref):
    u = u_ref[...]
    ln = len_ref[...]
    col = lax.broadcasted_iota(jnp.int32, u.shape, 1)
    u = jnp.where((col >= ln) | (col >= _C), -jnp.inf, u)
    m = jnp.max(u, axis=1, keepdims=True)
    sh = u - m
    lse = jnp.log(jnp.sum(jnp.exp(sh), axis=1, keepdims=True))
    o_ref[...] = (sh - lse)[:, :_C]


_BLK = 2048


def _tc_logsoftmax(util, lens2d):
    return pl.pallas_call(
        _softmax_body,
        grid=(_B // _BLK,),
        in_specs=[
            pl.BlockSpec((_BLK, _OP), lambda i: (i, 0)),
            pl.BlockSpec((_BLK, 1), lambda i: (i, 0)),
        ],
        out_specs=pl.BlockSpec((_BLK, _C), lambda i: (i, 0)),
        out_shape=jax.ShapeDtypeStruct((_B, _C), jnp.float32),
    )(util, lens2d)


def kernel(histories, history_lengths, choice_sets, choice_set_lengths,
           Wh, Wc, Wt):
    del history_lengths
    hidx = jnp.pad(histories, ((0, 0), (0, _HP - _H))).reshape(-1)
    cidx = jnp.pad(choice_sets, ((0, 0), (0, _CPD - _C))).reshape(-1)
    wh, wct = _repack_tables(Wh, Wc, Wt)
    util = _sc_utilities(hidx, cidx, wh, wct).reshape(_B, _OP)
    return _tc_logsoftmax(util, choice_set_lengths.reshape(_B, 1))
